# baseline (speedup 1.0000x reference)
import functools

import jax
import jax.numpy as jnp
from jax.experimental import pallas as pl
from jax.experimental.pallas import tpu as pltpu

_TEMP = 0.1
_BM = 256
_BN = 1024


def _loss_body(a_r, a_c, s_r, s_c, r_r, c_r, r_c, c_c, out,
               pos_acc, neg_acc, *, bm, bn):
    j = pl.program_id(1)
    nj = pl.num_programs(1)

    @pl.when(j == 0)
    def _():
        pos_acc[...] = jnp.zeros_like(pos_acc)
        neg_acc[...] = jnp.zeros_like(neg_acc)

    sim = jax.lax.dot_general(
        a_r[...], a_c[...], (((1,), (1,)), ((), ())),
        preferred_element_type=jnp.float32) / _TEMP

    score_mask = jnp.abs(s_r[...] - s_c[...]) < 0.1
    shared = ((r_r[...] == r_c[...]) | (r_r[...] == c_c[...])
              | (c_r[...] == r_c[...]) | (c_r[...] == c_c[...]))
    row_ids = pl.program_id(0) * bm + jax.lax.broadcasted_iota(
        jnp.int32, (bm, bn), 0)
    col_ids = j * bn + jax.lax.broadcasted_iota(jnp.int32, (bm, bn), 1)
    mask = score_mask | (shared & (row_ids != col_ids))

    esim = jnp.exp(sim)
    pos_acc[...] += jnp.sum(jnp.where(mask, sim, 0.0), axis=1, keepdims=True)
    neg_acc[...] += jnp.sum(jnp.where(mask, 0.0, esim), axis=1, keepdims=True)

    @pl.when(j == nj - 1)
    def _():
        p = pos_acc[...]
        n = neg_acc[...]
        out[...] = -jnp.log(p / (p + n + 1e-8))


@jax.jit
def kernel(edge_embeddings, edge_index, structural_scores):
    e, d = edge_embeddings.shape
    bm, bn = _BM, _BN
    row = edge_index[0].astype(jnp.int32)
    col = edge_index[1].astype(jnp.int32)
    s = structural_scores.astype(jnp.float32)

    grid = (e // bm, e // bn)
    loss_rows = pl.pallas_call(
        functools.partial(_loss_body, bm=bm, bn=bn),
        grid=grid,
        in_specs=[
            pl.BlockSpec((bm, d), lambda i, j: (i, 0)),
            pl.BlockSpec((bn, d), lambda i, j: (j, 0)),
            pl.BlockSpec((bm, 1), lambda i, j: (i, 0)),
            pl.BlockSpec((1, bn), lambda i, j: (0, j)),
            pl.BlockSpec((bm, 1), lambda i, j: (i, 0)),
            pl.BlockSpec((bm, 1), lambda i, j: (i, 0)),
            pl.BlockSpec((1, bn), lambda i, j: (0, j)),
            pl.BlockSpec((1, bn), lambda i, j: (0, j)),
        ],
        out_specs=pl.BlockSpec((bm, 1), lambda i, j: (i, 0)),
        out_shape=jax.ShapeDtypeStruct((e, 1), jnp.float32),
        scratch_shapes=[
            pltpu.VMEM((bm, 1), jnp.float32),
            pltpu.VMEM((bm, 1), jnp.float32),
        ],
        compiler_params=pltpu.CompilerParams(
            dimension_semantics=("parallel", "arbitrary"),
        ),
        name="contrastive_loss",
    )(
        edge_embeddings,
        edge_embeddings,
        s.reshape(e, 1),
        s.reshape(1, e),
        row.reshape(e, 1),
        col.reshape(e, 1),
        row.reshape(1, e),
        col.reshape(1, e),
    )
    return jnp.mean(loss_rows)


# speedup vs baseline: 1.5411x; 1.5411x over previous
import functools

import jax
import jax.numpy as jnp
from jax.experimental import pallas as pl
from jax.experimental.pallas import tpu as pltpu

_TEMP = 0.1
_BM = 512
_BN = 2048


def _loss_body(a_r, a_c, s_r, s_c, r_r, c_r, r_c, c_c, out,
               pos_acc, neg_acc, *, bm, bn):
    j = pl.program_id(1)
    nj = pl.num_programs(1)

    @pl.when(j == 0)
    def _():
        pos_acc[...] = jnp.zeros_like(pos_acc)
        neg_acc[...] = jnp.zeros_like(neg_acc)

    sim = jax.lax.dot_general(
        a_r[...], a_c[...], (((1,), (1,)), ((), ())),
        preferred_element_type=jnp.float32) / _TEMP

    score_mask = jnp.abs(s_r[...] - s_c[...]) < 0.1
    shared = ((r_r[...] == r_c[...]) | (r_r[...] == c_c[...])
              | (c_r[...] == r_c[...]) | (c_r[...] == c_c[...]))
    mask = score_mask | shared

    esim = jnp.exp(sim)
    pos_acc[...] += jnp.sum(jnp.where(mask, sim, 0.0), axis=1, keepdims=True)
    neg_acc[...] += jnp.sum(jnp.where(mask, 0.0, esim), axis=1, keepdims=True)

    @pl.when(j == nj - 1)
    def _():
        p = pos_acc[...]
        n = neg_acc[...]
        out[...] = -jnp.log(p / (p + n + 1e-8))


@jax.jit
def kernel(edge_embeddings, edge_index, structural_scores):
    e, d = edge_embeddings.shape
    bm, bn = _BM, _BN
    row = edge_index[0].astype(jnp.int32)
    col = edge_index[1].astype(jnp.int32)
    s = structural_scores.astype(jnp.float32)

    grid = (e // bm, e // bn)
    loss_rows = pl.pallas_call(
        functools.partial(_loss_body, bm=bm, bn=bn),
        grid=grid,
        in_specs=[
            pl.BlockSpec((bm, d), lambda i, j: (i, 0)),
            pl.BlockSpec((bn, d), lambda i, j: (j, 0)),
            pl.BlockSpec((bm, 1), lambda i, j: (i, 0)),
            pl.BlockSpec((1, bn), lambda i, j: (0, j)),
            pl.BlockSpec((bm, 1), lambda i, j: (i, 0)),
            pl.BlockSpec((bm, 1), lambda i, j: (i, 0)),
            pl.BlockSpec((1, bn), lambda i, j: (0, j)),
            pl.BlockSpec((1, bn), lambda i, j: (0, j)),
        ],
        out_specs=pl.BlockSpec((bm, 1), lambda i, j: (i, 0)),
        out_shape=jax.ShapeDtypeStruct((e, 1), jnp.float32),
        scratch_shapes=[
            pltpu.VMEM((bm, 1), jnp.float32),
            pltpu.VMEM((bm, 1), jnp.float32),
        ],
        compiler_params=pltpu.CompilerParams(
            dimension_semantics=("parallel", "arbitrary"),
        ),
        name="contrastive_loss",
    )(
        edge_embeddings,
        edge_embeddings,
        s.reshape(e, 1),
        s.reshape(1, e),
        row.reshape(e, 1),
        col.reshape(e, 1),
        row.reshape(1, e),
        col.reshape(1, e),
    )
    return jnp.mean(loss_rows)


# speedup vs baseline: 2.2066x; 1.4318x over previous
import functools

import jax
import jax.numpy as jnp
import numpy as np
from jax.experimental import pallas as pl
from jax.experimental.pallas import tpu as pltpu

_TEMP = 0.1
_B = 1024


def _tri_body(i_ref, j_ref, a_r, a_c, s_r, s_c, r_r, c_r, r_c, c_c,
              rp, rn, cp, cn, *, b):
    t = pl.program_id(0)
    is_diag = i_ref[t] == j_ref[t]

    sim = jax.lax.dot_general(
        a_r[...], a_c[...], (((1,), (1,)), ((), ())),
        preferred_element_type=jnp.float32) / _TEMP

    score_mask = jnp.abs(s_r[...] - s_c[...]) < 0.1
    shared = ((r_r[...] == r_c[...]) | (r_r[...] == c_c[...])
              | (c_r[...] == r_c[...]) | (c_r[...] == c_c[...]))
    mask = score_mask | shared

    esim = jnp.exp(sim)
    pos_t = jnp.where(mask, sim, 0.0)
    neg_t = jnp.where(mask, 0.0, esim)

    rp[...] = jnp.sum(pos_t, axis=1, keepdims=True)[None]
    rn[...] = jnp.sum(neg_t, axis=1, keepdims=True)[None]
    scale = jnp.where(is_diag, 0.0, 1.0)
    cp[...] = (jnp.sum(pos_t, axis=0, keepdims=True) * scale)[None]
    cn[...] = (jnp.sum(neg_t, axis=0, keepdims=True) * scale)[None]


def _row_sums(edge_embeddings, edge_index, structural_scores):
    e, d = edge_embeddings.shape
    b = _B
    nb = e // b
    row = edge_index[0].astype(jnp.int32)
    col = edge_index[1].astype(jnp.int32)
    s = structural_scores.astype(jnp.float32)

    iu, ju = np.triu_indices(nb)
    tt = len(iu)
    i_arr = jnp.asarray(iu, dtype=jnp.int32)
    j_arr = jnp.asarray(ju, dtype=jnp.int32)

    grid_spec = pltpu.PrefetchScalarGridSpec(
        num_scalar_prefetch=2,
        grid=(tt,),
        in_specs=[
            pl.BlockSpec((b, d), lambda t, i_r, j_r: (i_r[t], 0)),
            pl.BlockSpec((b, d), lambda t, i_r, j_r: (j_r[t], 0)),
            pl.BlockSpec((b, 1), lambda t, i_r, j_r: (i_r[t], 0)),
            pl.BlockSpec((1, b), lambda t, i_r, j_r: (0, j_r[t])),
            pl.BlockSpec((b, 1), lambda t, i_r, j_r: (i_r[t], 0)),
            pl.BlockSpec((b, 1), lambda t, i_r, j_r: (i_r[t], 0)),
            pl.BlockSpec((1, b), lambda t, i_r, j_r: (0, j_r[t])),
            pl.BlockSpec((1, b), lambda t, i_r, j_r: (0, j_r[t])),
        ],
        out_specs=[
            pl.BlockSpec((1, b, 1), lambda t, i_r, j_r: (t, 0, 0)),
            pl.BlockSpec((1, b, 1), lambda t, i_r, j_r: (t, 0, 0)),
            pl.BlockSpec((1, 1, b), lambda t, i_r, j_r: (t, 0, 0)),
            pl.BlockSpec((1, 1, b), lambda t, i_r, j_r: (t, 0, 0)),
        ],
    )
    rp, rn, cp, cn = pl.pallas_call(
        functools.partial(_tri_body, b=b),
        grid_spec=grid_spec,
        out_shape=[
            jax.ShapeDtypeStruct((tt, b, 1), jnp.float32),
            jax.ShapeDtypeStruct((tt, b, 1), jnp.float32),
            jax.ShapeDtypeStruct((tt, 1, b), jnp.float32),
            jax.ShapeDtypeStruct((tt, 1, b), jnp.float32),
        ],
        compiler_params=pltpu.CompilerParams(
            dimension_semantics=("parallel",),
        ),
        name="contrastive_loss_tri",
    )(
        i_arr, j_arr,
        edge_embeddings,
        edge_embeddings,
        s.reshape(e, 1),
        s.reshape(1, e),
        row.reshape(e, 1),
        col.reshape(e, 1),
        row.reshape(1, e),
        col.reshape(1, e),
    )

    pos = (jax.ops.segment_sum(rp[:, :, 0], i_arr, num_segments=nb)
           + jax.ops.segment_sum(cp[:, 0, :], j_arr, num_segments=nb)
           ).reshape(e)
    neg = (jax.ops.segment_sum(rn[:, :, 0], i_arr, num_segments=nb)
           + jax.ops.segment_sum(cn[:, 0, :], j_arr, num_segments=nb)
           ).reshape(e)
    return pos, neg


@jax.jit
def kernel(edge_embeddings, edge_index, structural_scores):
    pos, neg = _row_sums(edge_embeddings, edge_index, structural_scores)
    loss = -jnp.log(pos / (pos + neg + 1e-8))
    return jnp.mean(loss)


# speedup vs baseline: 2.2206x; 1.0064x over previous
import functools

import jax
import jax.numpy as jnp
import numpy as np
from jax.experimental import pallas as pl
from jax.experimental.pallas import tpu as pltpu

_TEMP = 0.1
_B = 1024


def _tri_body(i_ref, j_ref, a_r, a_c, s_r, r_r, c_r, s_j, ei_j,
              rp, rn, cp, cn, *, b):
    t = pl.program_id(0)
    is_diag = i_ref[t] == j_ref[t]

    sim = jax.lax.dot_general(
        a_r[...], a_c[...], (((1,), (1,)), ((), ())),
        preferred_element_type=jnp.float32) / _TEMP

    s_c = s_j[...]
    r_c = ei_j[0:1, :]
    c_c = ei_j[1:2, :]

    score_mask = jnp.abs(s_r[...] - s_c) < 0.1
    shared = ((r_r[...] == r_c) | (r_r[...] == c_c)
              | (c_r[...] == r_c) | (c_r[...] == c_c))
    mask = score_mask | shared

    esim = jnp.exp(sim)
    pos_t = jnp.where(mask, sim, 0.0)
    neg_t = jnp.where(mask, 0.0, esim)

    rp[...] = jnp.sum(pos_t, axis=1, keepdims=True)[None]
    rn[...] = jnp.sum(neg_t, axis=1, keepdims=True)[None]
    scale = jnp.where(is_diag, 0.0, 1.0)
    cp[...] = (jnp.sum(pos_t, axis=0, keepdims=True) * scale)[None]
    cn[...] = (jnp.sum(neg_t, axis=0, keepdims=True) * scale)[None]


def _partials(edge_embeddings, edge_index, structural_scores):
    e, d = edge_embeddings.shape
    b = _B
    nb = e // b
    ei = edge_index.astype(jnp.int32)
    s = structural_scores.astype(jnp.float32)

    iu, ju = np.triu_indices(nb)
    tt = len(iu)
    i_arr = jnp.asarray(iu, dtype=jnp.int32)
    j_arr = jnp.asarray(ju, dtype=jnp.int32)

    grid_spec = pltpu.PrefetchScalarGridSpec(
        num_scalar_prefetch=2,
        grid=(tt,),
        in_specs=[
            pl.BlockSpec((b, d), lambda t, i_r, j_r: (i_r[t], 0)),
            pl.BlockSpec((b, d), lambda t, i_r, j_r: (j_r[t], 0)),
            pl.BlockSpec((b, 1), lambda t, i_r, j_r: (i_r[t], 0)),
            pl.BlockSpec((b, 1), lambda t, i_r, j_r: (i_r[t], 0)),
            pl.BlockSpec((b, 1), lambda t, i_r, j_r: (i_r[t], 0)),
            pl.BlockSpec((1, b), lambda t, i_r, j_r: (0, j_r[t])),
            pl.BlockSpec((2, b), lambda t, i_r, j_r: (0, j_r[t])),
        ],
        out_specs=[
            pl.BlockSpec((1, b, 1), lambda t, i_r, j_r: (t, 0, 0)),
            pl.BlockSpec((1, b, 1), lambda t, i_r, j_r: (t, 0, 0)),
            pl.BlockSpec((1, 1, b), lambda t, i_r, j_r: (t, 0, 0)),
            pl.BlockSpec((1, 1, b), lambda t, i_r, j_r: (t, 0, 0)),
        ],
    )
    rp, rn, cp, cn = pl.pallas_call(
        functools.partial(_tri_body, b=b),
        grid_spec=grid_spec,
        out_shape=[
            jax.ShapeDtypeStruct((tt, b, 1), jnp.float32),
            jax.ShapeDtypeStruct((tt, b, 1), jnp.float32),
            jax.ShapeDtypeStruct((tt, 1, b), jnp.float32),
            jax.ShapeDtypeStruct((tt, 1, b), jnp.float32),
        ],
        compiler_params=pltpu.CompilerParams(
            dimension_semantics=("arbitrary",),
        ),
        name="contrastive_loss_tri",
    )(
        i_arr, j_arr,
        edge_embeddings,
        edge_embeddings,
        s.reshape(e, 1),
        ei[0].reshape(e, 1),
        ei[1].reshape(e, 1),
        s.reshape(1, e),
        ei,
    )

    return rp, rn, cp, cn, iu, ju, nb, e


def _row_sums(edge_embeddings, edge_index, structural_scores):
    rp, rn, cp, cn, iu, ju, nb, e = _partials(
        edge_embeddings, edge_index, structural_scores)
    i_arr = jnp.asarray(iu, dtype=jnp.int32)
    j_arr = jnp.asarray(ju, dtype=jnp.int32)
    pos = (jax.ops.segment_sum(rp[:, :, 0], i_arr, num_segments=nb)
           + jax.ops.segment_sum(cp[:, 0, :], j_arr, num_segments=nb)
           ).reshape(e)
    neg = (jax.ops.segment_sum(rn[:, :, 0], i_arr, num_segments=nb)
           + jax.ops.segment_sum(cn[:, 0, :], j_arr, num_segments=nb)
           ).reshape(e)
    return pos, neg


def _finalize_body(rp, rn, cp, cn, out, *, groups_i, groups_j, e):
    nb = len(groups_i)
    pos_rows = []
    neg_rows = []
    for blk in range(nb):
        p = jnp.zeros((1, rp.shape[1]), jnp.float32)
        n = jnp.zeros((1, rn.shape[1]), jnp.float32)
        for t in groups_i[blk]:
            p = p + rp[t:t + 1, :]
            n = n + rn[t:t + 1, :]
        for t in groups_j[blk]:
            p = p + cp[t:t + 1, :]
            n = n + cn[t:t + 1, :]
        pos_rows.append(p)
        neg_rows.append(n)
    pos = jnp.concatenate(pos_rows, axis=0)
    neg = jnp.concatenate(neg_rows, axis=0)
    loss = -jnp.log(pos / (pos + neg + 1e-8))
    out[0, 0] = jnp.sum(loss) / e


@jax.jit
def kernel(edge_embeddings, edge_index, structural_scores):
    rp, rn, cp, cn, iu, ju, nb, e = _partials(
        edge_embeddings, edge_index, structural_scores)
    iu = [int(x) for x in iu]
    ju = [int(x) for x in ju]
    groups_i = [[t for t, i in enumerate(iu) if i == blk] for blk in range(nb)]
    groups_j = [[t for t, (i, j) in enumerate(zip(iu, ju))
                 if j == blk and i != j] for blk in range(nb)]
    out = pl.pallas_call(
        functools.partial(_finalize_body, groups_i=groups_i,
                          groups_j=groups_j, e=e),
        out_specs=pl.BlockSpec((1, 1), memory_space=pltpu.SMEM),
        out_shape=jax.ShapeDtypeStruct((1, 1), jnp.float32),
        name="contrastive_loss_finalize",
    )(rp.reshape(rp.shape[0], -1), rn.reshape(rn.shape[0], -1),
      cp.reshape(cp.shape[0], -1), cn.reshape(cn.shape[0], -1))
    return out[0, 0]


# speedup vs baseline: 2.4158x; 1.0879x over previous
import functools

import jax
import jax.numpy as jnp
import numpy as np
from jax.experimental import pallas as pl
from jax.experimental.pallas import tpu as pltpu

_TEMP = 0.1
_B = 1024


def _tri_body(i_ref, j_ref, a_r, a_c, s_r, r_r, c_r, s_j, ei_j,
              rp, rn, cp, cn, *, b):
    t = pl.program_id(0)
    is_diag = i_ref[t] == j_ref[t]

    sim = jax.lax.dot_general(
        a_r[...], a_c[...], (((1,), (1,)), ((), ())),
        preferred_element_type=jnp.float32) / _TEMP

    s_c = s_j[...]
    r_c = ei_j[0:1, :]
    c_c = ei_j[1:2, :]

    score_mask = jnp.abs(s_r[...] - s_c) < 0.1
    shared = ((r_r[...] == r_c) | (r_r[...] == c_c)
              | (c_r[...] == r_c) | (c_r[...] == c_c))
    mask = score_mask | shared

    esim = jnp.exp(sim)
    pos_t = jnp.where(mask, sim, 0.0)
    neg_t = jnp.where(mask, 0.0, esim)

    rp[...] = jnp.swapaxes(jnp.sum(pos_t, axis=1, keepdims=True), 0, 1)[None]
    rn[...] = jnp.swapaxes(jnp.sum(neg_t, axis=1, keepdims=True), 0, 1)[None]
    scale = jnp.where(is_diag, 0.0, 1.0)
    cp[...] = (jnp.sum(pos_t, axis=0, keepdims=True) * scale)[None]
    cn[...] = (jnp.sum(neg_t, axis=0, keepdims=True) * scale)[None]


def _partials(edge_embeddings, edge_index, structural_scores):
    e, d = edge_embeddings.shape
    b = _B
    nb = e // b
    ei = edge_index.astype(jnp.int32)
    s = structural_scores.astype(jnp.float32)

    iu, ju = [], []
    fwd = True
    for i in range(nb):
        js = list(range(i, nb))
        if not fwd:
            js.reverse()
        fwd = not fwd
        iu.extend([i] * len(js))
        ju.extend(js)
    tt = len(iu)
    i_arr = jnp.asarray(iu, dtype=jnp.int32)
    j_arr = jnp.asarray(ju, dtype=jnp.int32)

    grid_spec = pltpu.PrefetchScalarGridSpec(
        num_scalar_prefetch=2,
        grid=(tt,),
        in_specs=[
            pl.BlockSpec((b, d), lambda t, i_r, j_r: (i_r[t], 0)),
            pl.BlockSpec((b, d), lambda t, i_r, j_r: (j_r[t], 0)),
            pl.BlockSpec((b, 1), lambda t, i_r, j_r: (i_r[t], 0)),
            pl.BlockSpec((b, 1), lambda t, i_r, j_r: (i_r[t], 0)),
            pl.BlockSpec((b, 1), lambda t, i_r, j_r: (i_r[t], 0)),
            pl.BlockSpec((1, b), lambda t, i_r, j_r: (0, j_r[t])),
            pl.BlockSpec((2, b), lambda t, i_r, j_r: (0, j_r[t])),
        ],
        out_specs=[
            pl.BlockSpec((1, 1, b), lambda t, i_r, j_r: (t, 0, 0)),
            pl.BlockSpec((1, 1, b), lambda t, i_r, j_r: (t, 0, 0)),
            pl.BlockSpec((1, 1, b), lambda t, i_r, j_r: (t, 0, 0)),
            pl.BlockSpec((1, 1, b), lambda t, i_r, j_r: (t, 0, 0)),
        ],
    )
    rp, rn, cp, cn = pl.pallas_call(
        functools.partial(_tri_body, b=b),
        grid_spec=grid_spec,
        out_shape=[
            jax.ShapeDtypeStruct((tt, 1, b), jnp.float32),
            jax.ShapeDtypeStruct((tt, 1, b), jnp.float32),
            jax.ShapeDtypeStruct((tt, 1, b), jnp.float32),
            jax.ShapeDtypeStruct((tt, 1, b), jnp.float32),
        ],
        compiler_params=pltpu.CompilerParams(
            dimension_semantics=("arbitrary",),
        ),
        name="contrastive_loss_tri",
    )(
        i_arr, j_arr,
        edge_embeddings,
        edge_embeddings,
        s.reshape(e, 1),
        ei[0].reshape(e, 1),
        ei[1].reshape(e, 1),
        s.reshape(1, e),
        ei,
    )

    return rp, rn, cp, cn, iu, ju, nb, e


def _row_sums(edge_embeddings, edge_index, structural_scores):
    rp, rn, cp, cn, iu, ju, nb, e = _partials(
        edge_embeddings, edge_index, structural_scores)
    i_arr = jnp.asarray(iu, dtype=jnp.int32)
    j_arr = jnp.asarray(ju, dtype=jnp.int32)
    pos = (jax.ops.segment_sum(rp[:, 0, :], i_arr, num_segments=nb)
           + jax.ops.segment_sum(cp[:, 0, :], j_arr, num_segments=nb)
           ).reshape(e)
    neg = (jax.ops.segment_sum(rn[:, 0, :], i_arr, num_segments=nb)
           + jax.ops.segment_sum(cn[:, 0, :], j_arr, num_segments=nb)
           ).reshape(e)
    return pos, neg


def _finalize_body(rp, rn, cp, cn, out, *, groups_i, groups_j, e):
    nb = len(groups_i)
    pos_rows = []
    neg_rows = []
    for blk in range(nb):
        p = jnp.zeros((1, rp.shape[1]), jnp.float32)
        n = jnp.zeros((1, rn.shape[1]), jnp.float32)
        for t in groups_i[blk]:
            p = p + rp[t:t + 1, :]
            n = n + rn[t:t + 1, :]
        for t in groups_j[blk]:
            p = p + cp[t:t + 1, :]
            n = n + cn[t:t + 1, :]
        pos_rows.append(p)
        neg_rows.append(n)
    pos = jnp.concatenate(pos_rows, axis=0)
    neg = jnp.concatenate(neg_rows, axis=0)
    loss = -jnp.log(pos / (pos + neg + 1e-8))
    out[0, 0] = jnp.sum(loss) / e


@jax.jit
def kernel(edge_embeddings, edge_index, structural_scores):
    rp, rn, cp, cn, iu, ju, nb, e = _partials(
        edge_embeddings, edge_index, structural_scores)
    iu = [int(x) for x in iu]
    ju = [int(x) for x in ju]
    groups_i = [[t for t, i in enumerate(iu) if i == blk] for blk in range(nb)]
    groups_j = [[t for t, (i, j) in enumerate(zip(iu, ju))
                 if j == blk and i != j] for blk in range(nb)]
    out = pl.pallas_call(
        functools.partial(_finalize_body, groups_i=groups_i,
                          groups_j=groups_j, e=e),
        out_specs=pl.BlockSpec((1, 1), memory_space=pltpu.SMEM),
        out_shape=jax.ShapeDtypeStruct((1, 1), jnp.float32),
        name="contrastive_loss_finalize",
    )(rp.reshape(rp.shape[0], -1), rn.reshape(rn.shape[0], -1),
      cp.reshape(cp.shape[0], -1), cn.reshape(cn.shape[0], -1))
    return out[0, 0]


# speedup vs baseline: 2.5578x; 1.0588x over previous
import functools

import jax
import jax.numpy as jnp
import numpy as np
from jax.experimental import pallas as pl
from jax.experimental.pallas import tpu as pltpu

_TEMP = 0.1
_B = 1024


def _tri_body(i_ref, j_ref, a_r, a_c, pk_i, s_j, ei_j,
              rp, rn, cp, cn, *, b):
    t = pl.program_id(0)
    is_diag = i_ref[t] == j_ref[t]

    sim = jax.lax.dot_general(
        a_r[...], a_c[...], (((1,), (1,)), ((), ())),
        preferred_element_type=jnp.float32) / _TEMP

    s_r = pk_i[:, 0:1]
    r_r = jax.lax.bitcast_convert_type(pk_i[:, 1:2], jnp.int32)
    c_r = jax.lax.bitcast_convert_type(pk_i[:, 2:3], jnp.int32)
    s_c = s_j[...]
    r_c = ei_j[0:1, :]
    c_c = ei_j[1:2, :]

    score_mask = jnp.abs(s_r - s_c) < 0.1
    shared = ((r_r == r_c) | (r_r == c_c)
              | (c_r == r_c) | (c_r == c_c))
    mask = score_mask | shared

    esim = jnp.exp(sim)
    pos_t = jnp.where(mask, sim, 0.0)
    neg_t = jnp.where(mask, 0.0, esim)

    rp[...] = jnp.swapaxes(jnp.sum(pos_t, axis=1, keepdims=True), 0, 1)[None]
    rn[...] = jnp.swapaxes(jnp.sum(neg_t, axis=1, keepdims=True), 0, 1)[None]
    scale = jnp.where(is_diag, 0.0, 1.0)
    cp[...] = (jnp.sum(pos_t, axis=0, keepdims=True) * scale)[None]
    cn[...] = (jnp.sum(neg_t, axis=0, keepdims=True) * scale)[None]


def _partials(edge_embeddings, edge_index, structural_scores):
    e, d = edge_embeddings.shape
    b = _B
    nb = e // b
    ei = edge_index.astype(jnp.int32)
    s = structural_scores.astype(jnp.float32)

    iu, ju = [], []
    fwd = True
    for i in range(nb):
        js = list(range(i, nb))
        if not fwd:
            js.reverse()
        fwd = not fwd
        iu.extend([i] * len(js))
        ju.extend(js)
    tt = len(iu)
    i_arr = jnp.asarray(iu, dtype=jnp.int32)
    j_arr = jnp.asarray(ju, dtype=jnp.int32)

    grid_spec = pltpu.PrefetchScalarGridSpec(
        num_scalar_prefetch=2,
        grid=(tt,),
        in_specs=[
            pl.BlockSpec((b, d), lambda t, i_r, j_r: (i_r[t], 0)),
            pl.BlockSpec((b, d), lambda t, i_r, j_r: (j_r[t], 0)),
            pl.BlockSpec((b, 3), lambda t, i_r, j_r: (i_r[t], 0)),
            pl.BlockSpec((1, b), lambda t, i_r, j_r: (0, j_r[t])),
            pl.BlockSpec((2, b), lambda t, i_r, j_r: (0, j_r[t])),
        ],
        out_specs=[
            pl.BlockSpec((1, 1, b), lambda t, i_r, j_r: (t, 0, 0)),
            pl.BlockSpec((1, 1, b), lambda t, i_r, j_r: (t, 0, 0)),
            pl.BlockSpec((1, 1, b), lambda t, i_r, j_r: (t, 0, 0)),
            pl.BlockSpec((1, 1, b), lambda t, i_r, j_r: (t, 0, 0)),
        ],
    )
    rp, rn, cp, cn = pl.pallas_call(
        functools.partial(_tri_body, b=b),
        grid_spec=grid_spec,
        out_shape=[
            jax.ShapeDtypeStruct((tt, 1, b), jnp.float32),
            jax.ShapeDtypeStruct((tt, 1, b), jnp.float32),
            jax.ShapeDtypeStruct((tt, 1, b), jnp.float32),
            jax.ShapeDtypeStruct((tt, 1, b), jnp.float32),
        ],
        compiler_params=pltpu.CompilerParams(
            dimension_semantics=("arbitrary",),
        ),
        name="contrastive_loss_tri",
    )(
        i_arr, j_arr,
        edge_embeddings,
        edge_embeddings,
        jnp.concatenate(
            [s.reshape(e, 1),
             jax.lax.bitcast_convert_type(ei[0], jnp.float32).reshape(e, 1),
             jax.lax.bitcast_convert_type(ei[1], jnp.float32).reshape(e, 1)],
            axis=1),
        s.reshape(1, e),
        ei,
    )

    return rp, rn, cp, cn, iu, ju, nb, e


def _row_sums(edge_embeddings, edge_index, structural_scores):
    rp, rn, cp, cn, iu, ju, nb, e = _partials(
        edge_embeddings, edge_index, structural_scores)
    i_arr = jnp.asarray(iu, dtype=jnp.int32)
    j_arr = jnp.asarray(ju, dtype=jnp.int32)
    pos = (jax.ops.segment_sum(rp[:, 0, :], i_arr, num_segments=nb)
           + jax.ops.segment_sum(cp[:, 0, :], j_arr, num_segments=nb)
           ).reshape(e)
    neg = (jax.ops.segment_sum(rn[:, 0, :], i_arr, num_segments=nb)
           + jax.ops.segment_sum(cn[:, 0, :], j_arr, num_segments=nb)
           ).reshape(e)
    return pos, neg


def _finalize_body(rp, rn, cp, cn, out, *, groups_i, groups_j, e):
    nb = len(groups_i)
    pos_rows = []
    neg_rows = []
    for blk in range(nb):
        p = jnp.zeros((1, rp.shape[1]), jnp.float32)
        n = jnp.zeros((1, rn.shape[1]), jnp.float32)
        for t in groups_i[blk]:
            p = p + rp[t:t + 1, :]
            n = n + rn[t:t + 1, :]
        for t in groups_j[blk]:
            p = p + cp[t:t + 1, :]
            n = n + cn[t:t + 1, :]
        pos_rows.append(p)
        neg_rows.append(n)
    pos = jnp.concatenate(pos_rows, axis=0)
    neg = jnp.concatenate(neg_rows, axis=0)
    loss = -jnp.log(pos / (pos + neg + 1e-8))
    out[0, 0] = jnp.sum(loss) / e


@jax.jit
def kernel(edge_embeddings, edge_index, structural_scores):
    rp, rn, cp, cn, iu, ju, nb, e = _partials(
        edge_embeddings, edge_index, structural_scores)
    iu = [int(x) for x in iu]
    ju = [int(x) for x in ju]
    groups_i = [[t for t, i in enumerate(iu) if i == blk] for blk in range(nb)]
    groups_j = [[t for t, (i, j) in enumerate(zip(iu, ju))
                 if j == blk and i != j] for blk in range(nb)]
    out = pl.pallas_call(
        functools.partial(_finalize_body, groups_i=groups_i,
                          groups_j=groups_j, e=e),
        out_specs=pl.BlockSpec((1, 1), memory_space=pltpu.SMEM),
        out_shape=jax.ShapeDtypeStruct((1, 1), jnp.float32),
        name="contrastive_loss_finalize",
    )(rp.reshape(rp.shape[0], -1), rn.reshape(rn.shape[0], -1),
      cp.reshape(cp.shape[0], -1), cn.reshape(cn.shape[0], -1))
    return out[0, 0]
